# baseline (speedup 1.0000x reference)
import functools

import jax
import jax.numpy as jnp
from jax import lax
from jax.experimental import pallas as pl
from jax.experimental.pallas import tpu as pltpu
from jax.experimental.pallas import tpu_sc as plsc

_B, _N, _C, _H, _P, _L = 16, 32, 8, 256, 64, 8
_NPATH = _B * _P
_EDGES = _L - 1
_NW = 32
_PPW = _NPATH // _NW
_EPW = _PPW * _EDGES
_NCHUNK = _EPW // 16

def _sc_body(table_hbm, paths_hbm, comm_hbm, out_hbm,
             paths_v, comm_v, ids_v, rows_v, out_v, sem):
    wid = lax.axis_index("s") * 2 + lax.axis_index("c")
    b = wid // 2
    pltpu.sync_copy(paths_hbm.at[pl.ds(wid * _PPW * _L, _PPW * _L)], paths_v)
    pltpu.sync_copy(comm_hbm, comm_v)
    c_vec = plsc.load_gather(comm_v, [jnp.full((16,), b, jnp.int32)])
    bN = (b * _N).astype(jnp.int32)
    lane = lax.iota(jnp.int32, 16)

    def fill_ids(k, half):
        jj = lane + (16 * k)
        idx_u = jj + jj // _EDGES
        u = plsc.load_gather(paths_v, [idx_u])
        v = plsc.load_gather(paths_v, [idx_u + 1])
        ids = ((bN + u) * _N + v) * _C + c_vec
        ids_v[half, pl.ds((16 * k) % (_EPW // 2), 16)] = ids

    for k in range(_NCHUNK // 2):
        fill_ids(k, 0)
    cp0 = pltpu.async_copy(table_hbm.at[ids_v.at[0]],
                           rows_v.at[pl.ds(0, _EPW // 2)], sem)
    for k in range(_NCHUNK // 2, _NCHUNK):
        fill_ids(k, 1)
    cp1 = pltpu.async_copy(table_hbm.at[ids_v.at[1]],
                           rows_v.at[pl.ds(_EPW // 2, _EPW // 2)], sem)

    def mean_one_path(p, carry):
        for h in range(_H // 16):
            sl = pl.ds(16 * h, 16)
            r = [rows_v[p * _EDGES + e, sl] for e in range(_EDGES)]
            acc = ((r[0] + r[1]) + (r[2] + r[3])) + ((r[4] + r[5]) + r[6])
            out_v[p, sl] = acc * (1.0 / _EDGES)
        return carry

    cp0.wait()
    lax.fori_loop(0, _PPW // 2, mean_one_path, 0)
    cp1.wait()
    lax.fori_loop(_PPW // 2, _PPW, mean_one_path, 0)
    pltpu.sync_copy(out_v, out_hbm.at[pl.ds(wid * _PPW, _PPW)])


@functools.cache
def _sc_gather_mean():
    return pl.kernel(
        _sc_body,
        mesh=plsc.VectorSubcoreMesh(core_axis_name="c", subcore_axis_name="s"),
        compiler_params=pltpu.CompilerParams(needs_layout_passes=False),
        out_type=jax.ShapeDtypeStruct((_NPATH, _H), jnp.float32),
        scratch_types=[
            pltpu.VMEM((_PPW * _L,), jnp.int32),
            pltpu.VMEM((_B,), jnp.int32),
            pltpu.VMEM((2, _EPW // 2), jnp.int32),
            pltpu.VMEM((_EPW, _H), jnp.float32),
            pltpu.VMEM((_PPW, _H), jnp.float32),
            pltpu.SemaphoreType.DMA,
        ],
    )


def _tc_body(pf_ref, g_ref, w1_ref, b1_ref, w2_ref, b2_ref, mask_ref,
             probs_ref, logp_ref, ent_ref):
    pf = pf_ref[...]
    h = jnp.dot(pf, w1_ref[0:_H, :], preferred_element_type=jnp.float32)
    hg = jnp.dot(g_ref[...], w1_ref[_H:2 * _H, :],
                 preferred_element_type=jnp.float32)
    hg = hg + b1_ref[...]
    h = h.reshape(_B, _P, 128) + hg[:, None, :]
    h = jnp.maximum(h, 0.0)
    w2 = w2_ref[...]
    scores = jnp.sum(h * w2[None, :, :], axis=-1) + b2_ref[0, 0]
    mask = mask_ref[...] != 0
    scores = jnp.where(mask, scores, -jnp.inf)
    m = jnp.max(scores, axis=-1, keepdims=True)
    ex = jnp.exp(scores - m)
    s = jnp.sum(ex, axis=-1, keepdims=True)
    probs = ex / s
    logp = (scores - m) - jnp.log(s)
    logp_safe = jnp.where(mask, logp, 0.0)
    ent = -jnp.sum(probs * logp_safe, axis=-1, keepdims=True)
    probs_ref[...] = probs
    logp_ref[...] = logp
    ent_ref[...] = ent


_tc_mlp_softmax = pl.pallas_call(
    _tc_body,
    out_shape=(
        jax.ShapeDtypeStruct((_B, _P), jnp.float32),
        jax.ShapeDtypeStruct((_B, _P), jnp.float32),
        jax.ShapeDtypeStruct((_B, 1), jnp.float32),
    ),
)


def kernel(edge_features, graph_embedding, selected_commodity, candidate_paths,
           path_mask, W1, b1, W2, b2):
    table = edge_features.reshape(_B * _N * _N * _C, _H)
    paths_flat = candidate_paths.astype(jnp.int32).reshape(_NPATH * _L)
    comm = selected_commodity.astype(jnp.int32)

    path_feat = _sc_gather_mean()(table, paths_flat, comm)

    probs, logp, ent = _tc_mlp_softmax(
        path_feat,
        graph_embedding,
        W1,
        b1.reshape(1, 128),
        W2.reshape(1, 128),
        b2.reshape(1, 1),
        path_mask.astype(jnp.int32),
    )
    return (probs, logp, ent.reshape(_B))


# speedup vs baseline: 1.0483x; 1.0483x over previous
import functools

import jax
import jax.numpy as jnp
from jax import lax
from jax.experimental import pallas as pl
from jax.experimental.pallas import tpu as pltpu
from jax.experimental.pallas import tpu_sc as plsc

_B, _N, _C, _H, _P, _L = 16, 32, 8, 256, 64, 8
_NPATH = _B * _P
_EDGES = _L - 1
_NW = 32
_PPW = _NPATH // _NW
_EPW = _PPW * _EDGES
_HALF = _EPW // 2


def _sc_body(table_hbm, paths_hbm, comm_hbm, out_hbm,
             paths_v, comm_v, ids_v, rows_v, out_v, sem, sem_out):
    wid = lax.axis_index("s") * 2 + lax.axis_index("c")
    b = wid // 2
    p0 = (wid % 2) * _PPW
    pltpu.sync_copy(paths_hbm.at[b], paths_v)
    pltpu.sync_copy(comm_hbm, comm_v)
    c_vec = plsc.load_gather(comm_v, [jnp.full((16,), b, jnp.int32)])
    bN = (b * _N).astype(jnp.int32)
    lane = lax.iota(jnp.int32, 16)

    def fill_ids(c):
        n = [paths_v[l, pl.ds(p0 + 16 * c, 16)] for l in range(_L)]
        for l in range(_EDGES):
            ids = ((bN + n[l]) * _N + n[l + 1]) * _C + c_vec
            plsc.store_scatter(ids_v, [jnp.full((16,), c, jnp.int32),
                                       lane * _EDGES + l], ids)

    fill_ids(0)
    cp0 = pltpu.async_copy(table_hbm.at[ids_v.at[0]],
                           rows_v.at[pl.ds(0, _HALF)], sem)
    fill_ids(1)
    cp1 = pltpu.async_copy(table_hbm.at[ids_v.at[1]],
                           rows_v.at[pl.ds(_HALF, _HALF)], sem)

    def mean_one_path(p, carry):
        for h in range(_H // 16):
            sl = pl.ds(16 * h, 16)
            r = [rows_v[p * _EDGES + e, sl] for e in range(_EDGES)]
            acc = ((r[0] + r[1]) + (r[2] + r[3])) + ((r[4] + r[5]) + r[6])
            out_v[p, sl] = acc * (1.0 / _EDGES)
        return carry

    cp0.wait()
    lax.fori_loop(0, _PPW // 2, mean_one_path, 0)
    cp_out0 = pltpu.async_copy(out_v.at[pl.ds(0, _PPW // 2)],
                               out_hbm.at[pl.ds(wid * _PPW, _PPW // 2)],
                               sem_out)
    cp1.wait()
    lax.fori_loop(_PPW // 2, _PPW, mean_one_path, 0)
    pltpu.sync_copy(out_v.at[pl.ds(_PPW // 2, _PPW // 2)],
                    out_hbm.at[pl.ds(wid * _PPW + _PPW // 2, _PPW // 2)])
    cp_out0.wait()


@functools.cache
def _sc_gather_mean():
    return pl.kernel(
        _sc_body,
        mesh=plsc.VectorSubcoreMesh(core_axis_name="c", subcore_axis_name="s"),
        compiler_params=pltpu.CompilerParams(needs_layout_passes=False),
        out_type=jax.ShapeDtypeStruct((_NPATH, _H), jnp.float32),
        scratch_types=[
            pltpu.VMEM((_L, _P), jnp.int32),
            pltpu.VMEM((_B,), jnp.int32),
            pltpu.VMEM((2, _HALF), jnp.int32),
            pltpu.VMEM((_EPW, _H), jnp.float32),
            pltpu.VMEM((_PPW, _H), jnp.float32),
            pltpu.SemaphoreType.DMA,
            pltpu.SemaphoreType.DMA,
        ],
    )


def _tc_body(pf_ref, g_ref, w1_ref, b1_ref, w2_ref, b2_ref, mask_ref,
             probs_ref, logp_ref, ent_ref):
    pf = pf_ref[...]
    h = jnp.dot(pf, w1_ref[0:_H, :], preferred_element_type=jnp.float32)
    hg = jnp.dot(g_ref[...], w1_ref[_H:2 * _H, :],
                 preferred_element_type=jnp.float32)
    hg = hg + b1_ref[...]
    h = h.reshape(_B, _P, 128) + hg[:, None, :]
    h = jnp.maximum(h, 0.0)
    w2 = w2_ref[...]
    scores = jnp.sum(h * w2[None, :, :], axis=-1) + b2_ref[0, 0]
    mask = mask_ref[...] != 0
    scores = jnp.where(mask, scores, -jnp.inf)
    m = jnp.max(scores, axis=-1, keepdims=True)
    ex = jnp.exp(scores - m)
    s = jnp.sum(ex, axis=-1, keepdims=True)
    probs = ex / s
    logp = (scores - m) - jnp.log(s)
    logp_safe = jnp.where(mask, logp, 0.0)
    ent = -jnp.sum(probs * logp_safe, axis=-1)
    probs_ref[...] = probs
    logp_ref[...] = logp
    ent_ref[...] = ent


_tc_mlp_softmax = pl.pallas_call(
    _tc_body,
    out_shape=(
        jax.ShapeDtypeStruct((_B, _P), jnp.float32),
        jax.ShapeDtypeStruct((_B, _P), jnp.float32),
        jax.ShapeDtypeStruct((_B,), jnp.float32),
    ),
)


def kernel(edge_features, graph_embedding, selected_commodity, candidate_paths,
           path_mask, W1, b1, W2, b2):
    table = edge_features.reshape(_B * _N * _N * _C, _H)
    paths_t = jnp.transpose(candidate_paths.astype(jnp.int32), (0, 2, 1))
    comm = selected_commodity.astype(jnp.int32)

    path_feat = _sc_gather_mean()(table, paths_t, comm)

    probs, logp, ent = _tc_mlp_softmax(
        path_feat,
        graph_embedding,
        W1,
        b1.reshape(1, 128),
        W2.reshape(1, 128),
        b2.reshape(1, 1),
        path_mask.astype(jnp.int32),
    )
    return (probs, logp, ent)
